# baseline (speedup 1.0000x reference)
import dataclasses
import functools

import jax
import jax.numpy as jnp
from jax import lax
from jax.experimental import pallas as pl
from jax.experimental.pallas import tpu as pltpu
from jax.experimental.pallas import tpu_sc as plsc

N = 10000
E = 320000
D = 128
NH = 64
NC = 2
NS = 16
K = 128
FIRE = 2
IW = 32
CH = 160
EPAD = NS * CH * K
RPT = 640
RPAD = NS * RPT
DCH = 80
DEG_P = 10016

_mesh = plsc.VectorSubcoreMesh(core_axis_name="c", subcore_axis_name="s")

_sc_params = pltpu.CompilerParams()
if "needs_layout_passes" in pltpu.CompilerParams.__dataclass_fields__:
    _sc_params = dataclasses.replace(_sc_params, needs_layout_passes=False)
if "use_tc_tiling_on_sc" in pltpu.CompilerParams.__dataclass_fields__:
    _sc_params = dataclasses.replace(_sc_params, use_tc_tiling_on_sc=False)


@functools.partial(
    pl.kernel,
    out_type=jax.ShapeDtypeStruct((NC * NS, DEG_P), jnp.float32),
    mesh=_mesh,
    compiler_params=_sc_params,
    scratch_types=[
        pltpu.VMEM((DCH, K), jnp.int32),
        pltpu.VMEM((DEG_P,), jnp.float32),
    ],
)
def _deg_kernel(dst_hbm, deg_hbm, dst_v, hist_v):
    w = lax.axis_index("c") * NS + lax.axis_index("s")
    pltpu.sync_copy(dst_hbm.at[w], dst_v)

    @pl.loop(0, DEG_P // 16)
    def _(i):
        hist_v[pl.ds(i * 16, 16)] = jnp.zeros((16,), jnp.float32)

    ones = jnp.ones((16,), jnp.float32)

    @pl.loop(0, DCH)
    def _(cc):
        for j in range(K // 16):
            idx = dst_v[cc, pl.ds(j * 16, 16)]
            plsc.addupdate_scatter(hist_v, [idx], ones)

    pltpu.sync_copy(hist_v, deg_hbm.at[w])


@functools.partial(
    pl.kernel,
    out_type=jax.ShapeDtypeStruct((NC, RPAD, NH), jnp.float32),
    mesh=_mesh,
    compiler_params=_sc_params,
    scratch_types=[
        pltpu.VMEM((IW, K), jnp.int32),
        pltpu.VMEM((IW, K), jnp.int32),
        pltpu.VMEM((2 * FIRE * K, NH), jnp.float32),
        pltpu.VMEM_SHARED((RPAD, NH), jnp.float32),
        pltpu.VMEM_SHARED((N, NH), jnp.float32),
        pltpu.SemaphoreType.DMA,
        pltpu.SemaphoreType.DMA,
        pltpu.SemaphoreType.DMA,
    ],
)
def _agg_kernel(h0_hbm, h1_hbm, src_hbm, dst_hbm, out_hbm, src_v, dst_v,
                rows_v, acc_sh, h_sh, gsem, hsem, ssem):
    c = lax.axis_index("c")
    s = lax.axis_index("s")

    HRT = N // NS

    @pl.when(c == 0)
    def _():
        pltpu.sync_copy(h0_hbm.at[pl.ds(s * HRT, HRT)], h_sh.at[pl.ds(s * HRT, HRT)])

    @pl.when(c == 1)
    def _():
        pltpu.sync_copy(h1_hbm.at[pl.ds(s * HRT, HRT)], h_sh.at[pl.ds(s * HRT, HRT)])

    @pl.loop(0, 2 * FIRE * K)
    def _(r):
        for j in range(NH // 16):
            rows_v[r, pl.ds(j * 16, 16)] = jnp.zeros((16,), jnp.float32)

    pltpu.sync_copy(rows_v, acc_sh.at[pl.ds(s * RPT, 2 * FIRE * K)])
    pltpu.sync_copy(
        rows_v.at[pl.ds(0, RPT - 2 * FIRE * K)],
        acc_sh.at[pl.ds(s * RPT + 2 * FIRE * K, RPT - 2 * FIRE * K)],
    )
    plsc.subcore_barrier()

    def pipeline(h_hbm):
        @pl.loop(0, CH, step=IW)
        def _(w0):
            pltpu.sync_copy(src_hbm.at[s, pl.ds(w0, IW)], src_v)
            pltpu.sync_copy(dst_hbm.at[s, pl.ds(w0, IW)], dst_v)

            @pl.loop(0, IW, step=2 * FIRE)
            def _(cl):
                g = []
                for q in range(2 * FIRE):
                    spmem_src = q % 2 == 0
                    g.append(
                        pltpu.async_copy(
                            (h_sh if spmem_src else h_hbm).at[src_v.at[cl + q]],
                            rows_v.at[pl.ds(q * K, K)],
                            gsem if spmem_src else hsem,
                        )
                    )
                sc = []
                for half in range(2):
                    for b in range(FIRE):
                        g[half * FIRE + b].wait()
                    for b in range(FIRE):
                        q = half * FIRE + b
                        sc.append(
                            pltpu.async_copy(
                                rows_v.at[pl.ds(q * K, K)],
                                acc_sh.at[dst_v.at[cl + q]],
                                ssem,
                                add=True,
                            )
                        )
                for cp in sc:
                    cp.wait()

    @pl.when(c == 0)
    def _():
        pipeline(h0_hbm)

    @pl.when(c == 1)
    def _():
        pipeline(h1_hbm)

    plsc.subcore_barrier()
    pltpu.sync_copy(
        acc_sh.at[pl.ds(s * RPT, RPT)],
        out_hbm.at[c, pl.ds(s * RPT, RPT)],
    )


BR = 2000
NB = N // BR


def _dinv_body(deg_ref, dinv_ref):
    deg = jnp.sum(deg_ref[...], axis=0)[:N] + 1.0
    dinv_ref[...] = lax.rsqrt(deg)[:, None]


_dinv = pl.pallas_call(
    _dinv_body,
    out_shape=jax.ShapeDtypeStruct((N, 1), jnp.float32),
)


def _scale_body(dinv_ref, x_ref, w_ref, h0_ref, h1_ref):
    h = jnp.dot(
        x_ref[...], w_ref[...],
        preferred_element_type=jnp.float32,
        precision=lax.Precision.HIGHEST,
    ) * dinv_ref[...]
    h0_ref[...] = h[:, :NH]
    h1_ref[...] = h[:, NH:]


_scale = pl.pallas_call(
    _scale_body,
    grid=(NB,),
    in_specs=[
        pl.BlockSpec((BR, 1), lambda i: (i, 0)),
        pl.BlockSpec((BR, D), lambda i: (i, 0)),
        pl.BlockSpec((D, D), lambda i: (0, 0)),
    ],
    out_specs=[
        pl.BlockSpec((BR, NH), lambda i: (i, 0)),
        pl.BlockSpec((BR, NH), lambda i: (i, 0)),
    ],
    out_shape=[
        jax.ShapeDtypeStruct((N, NH), jnp.float32),
        jax.ShapeDtypeStruct((N, NH), jnp.float32),
    ],
)


def _mid_body(p_ref, h0_ref, h1_ref, dinv_ref, b1_ref, w2_ref,
              h20_ref, h21_ref):
    agg = jnp.concatenate(
        [p_ref[0] + h0_ref[...], p_ref[1] + h1_ref[...]],
        axis=1,
    )
    z = jnp.maximum(agg * dinv_ref[...] + b1_ref[...][None, :], 0.0)
    h2 = jnp.dot(
        z, w2_ref[...],
        preferred_element_type=jnp.float32,
        precision=lax.Precision.HIGHEST,
    ) * dinv_ref[...]
    h20_ref[...] = h2[:, :NH]
    h21_ref[...] = h2[:, NH:]


_mid = pl.pallas_call(
    _mid_body,
    grid=(NB,),
    in_specs=[
        pl.BlockSpec((NC, BR, NH), lambda i: (0, i, 0)),
        pl.BlockSpec((BR, NH), lambda i: (i, 0)),
        pl.BlockSpec((BR, NH), lambda i: (i, 0)),
        pl.BlockSpec((BR, 1), lambda i: (i, 0)),
        pl.BlockSpec((D,), lambda i: (0,)),
        pl.BlockSpec((D, D), lambda i: (0, 0)),
    ],
    out_specs=[
        pl.BlockSpec((BR, NH), lambda i: (i, 0)),
        pl.BlockSpec((BR, NH), lambda i: (i, 0)),
    ],
    out_shape=[
        jax.ShapeDtypeStruct((N, NH), jnp.float32),
        jax.ShapeDtypeStruct((N, NH), jnp.float32),
    ],
)


def _out_body(p_ref, h20_ref, h21_ref, dinv_ref, b2_ref, o_ref):
    agg = jnp.concatenate(
        [p_ref[0] + h20_ref[...], p_ref[1] + h21_ref[...]],
        axis=1,
    )
    o_ref[...] = agg * dinv_ref[...] + b2_ref[...][None, :]


_out = pl.pallas_call(
    _out_body,
    grid=(NB,),
    in_specs=[
        pl.BlockSpec((NC, BR, NH), lambda i: (0, i, 0)),
        pl.BlockSpec((BR, NH), lambda i: (i, 0)),
        pl.BlockSpec((BR, NH), lambda i: (i, 0)),
        pl.BlockSpec((BR, 1), lambda i: (i, 0)),
        pl.BlockSpec((D,), lambda i: (0,)),
    ],
    out_specs=pl.BlockSpec((BR, D), lambda i: (i, 0)),
    out_shape=jax.ShapeDtypeStruct((N, D), jnp.float32),
)


@jax.jit
def kernel(x, edge_index, W1, b1, W2, b2):
    src = edge_index[0].astype(jnp.int32)
    dst = edge_index[1].astype(jnp.int32)
    pad = EPAD - E
    srcp = jnp.concatenate([src, jnp.zeros((pad,), jnp.int32)])
    dstp = jnp.concatenate([dst, jnp.full((pad,), N, jnp.int32)])
    srcp16 = srcp.reshape(NS, CH, K)
    dstp16 = dstp.reshape(NS, CH, K)
    dstp32 = dstp.reshape(NC * NS, DCH, K)

    deg = _deg_kernel(dstp32)
    dinv = _dinv(deg)
    h0, h1 = _scale(dinv, x, W1)
    p1 = _agg_kernel(h0, h1, srcp16, dstp16)
    h20, h21 = _mid(p1, h0, h1, dinv, b1, W2)
    p2 = _agg_kernel(h20, h21, srcp16, dstp16)
    return _out(p2, h20, h21, dinv, b2)


# speedup vs baseline: 1.1872x; 1.1872x over previous
import dataclasses
import functools

import jax
import jax.numpy as jnp
from jax import lax
from jax.experimental import pallas as pl
from jax.experimental.pallas import tpu as pltpu
from jax.experimental.pallas import tpu_sc as plsc

N = 10000
E = 320000
D = 128
NH = 64
NC = 2
NS = 16
K = 128
FIRE = 2
IW = 32
CH = 160
EPAD = NS * CH * K
RPT = 640
RPAD = NS * RPT
DCH = 80
DEG_P = 10240
NWIN = CH // IW

_mesh = plsc.VectorSubcoreMesh(core_axis_name="c", subcore_axis_name="s")

_sc_params = pltpu.CompilerParams()
if "needs_layout_passes" in pltpu.CompilerParams.__dataclass_fields__:
    _sc_params = dataclasses.replace(_sc_params, needs_layout_passes=False)
if "use_tc_tiling_on_sc" in pltpu.CompilerParams.__dataclass_fields__:
    _sc_params = dataclasses.replace(_sc_params, use_tc_tiling_on_sc=False)


@functools.partial(
    pl.kernel,
    out_type=jax.ShapeDtypeStruct((NC * NS, DEG_P), jnp.float32),
    mesh=_mesh,
    compiler_params=_sc_params,
    scratch_types=[
        pltpu.VMEM((DCH, K), jnp.int32),
        pltpu.VMEM((DEG_P,), jnp.float32),
    ],
)
def _deg_kernel(dst_hbm, deg_hbm, dst_v, hist_v):
    w = lax.axis_index("c") * NS + lax.axis_index("s")
    pltpu.sync_copy(dst_hbm.at[w], dst_v)

    @pl.loop(0, DEG_P // 16)
    def _(i):
        hist_v[pl.ds(i * 16, 16)] = jnp.zeros((16,), jnp.float32)

    ones = jnp.ones((16,), jnp.float32)

    @pl.loop(0, DCH)
    def _(cc):
        for j in range(K // 16):
            idx = dst_v[cc, pl.ds(j * 16, 16)]
            plsc.addupdate_scatter(hist_v, [idx], ones)

    pltpu.sync_copy(hist_v, deg_hbm.at[w])


@functools.partial(
    pl.kernel,
    out_type=jax.ShapeDtypeStruct((NC, RPAD, NH), jnp.float32),
    mesh=_mesh,
    compiler_params=_sc_params,
    scratch_types=[
        pltpu.VMEM((2, IW, K), jnp.int32),
        pltpu.VMEM((2, IW, K), jnp.int32),
        pltpu.VMEM((2 * FIRE * K, NH), jnp.float32),
        pltpu.VMEM_SHARED((RPAD, NH), jnp.float32),
        pltpu.VMEM_SHARED((N, NH), jnp.float32),
        pltpu.SemaphoreType.DMA,
        pltpu.SemaphoreType.DMA,
        pltpu.SemaphoreType.DMA,
    ],
)
def _agg_kernel(h0_hbm, h1_hbm, src_hbm, dst_hbm, out_hbm, src_v, dst_v,
                rows_v, acc_sh, h_sh, gsem, hsem, ssem):
    c = lax.axis_index("c")
    s = lax.axis_index("s")

    HRT = N // NS

    @pl.when(c == 0)
    def _():
        pltpu.sync_copy(h0_hbm.at[pl.ds(s * HRT, HRT)], h_sh.at[pl.ds(s * HRT, HRT)])

    @pl.when(c == 1)
    def _():
        pltpu.sync_copy(h1_hbm.at[pl.ds(s * HRT, HRT)], h_sh.at[pl.ds(s * HRT, HRT)])

    @pl.loop(0, 2 * FIRE * K)
    def _(r):
        for j in range(NH // 16):
            rows_v[r, pl.ds(j * 16, 16)] = jnp.zeros((16,), jnp.float32)

    pltpu.sync_copy(rows_v, acc_sh.at[pl.ds(s * RPT, 2 * FIRE * K)])
    pltpu.sync_copy(
        rows_v.at[pl.ds(0, RPT - 2 * FIRE * K)],
        acc_sh.at[pl.ds(s * RPT + 2 * FIRE * K, RPT - 2 * FIRE * K)],
    )
    plsc.subcore_barrier()

    pltpu.sync_copy(src_hbm.at[s, pl.ds(0, IW)], src_v.at[0])
    pltpu.sync_copy(dst_hbm.at[s, pl.ds(0, IW)], dst_v.at[0])
    pending = None
    for t in range(NWIN):
        slot = t % 2
        if pending is not None:
            for cp in pending:
                cp.wait()
        if t + 1 < NWIN:
            nslot = (t + 1) % 2
            pending = [
                pltpu.async_copy(
                    src_hbm.at[s, pl.ds((t + 1) * IW, IW)], src_v.at[nslot],
                    hsem,
                ),
                pltpu.async_copy(
                    dst_hbm.at[s, pl.ds((t + 1) * IW, IW)], dst_v.at[nslot],
                    hsem,
                ),
            ]
        else:
            pending = None

        @pl.loop(0, IW, step=2 * FIRE)
        def _(cl):
            g = []
            for q in range(2 * FIRE):
                g.append(
                    pltpu.async_copy(
                        h_sh.at[src_v.at[slot, cl + q]],
                        rows_v.at[pl.ds(q * K, K)],
                        gsem,
                    )
                )
            sc = []
            for half in range(2):
                for b in range(FIRE):
                    g[half * FIRE + b].wait()
                for b in range(FIRE):
                    q = half * FIRE + b
                    sc.append(
                        pltpu.async_copy(
                            rows_v.at[pl.ds(q * K, K)],
                            acc_sh.at[dst_v.at[slot, cl + q]],
                            ssem,
                            add=True,
                        )
                    )
            for cp in sc:
                cp.wait()

    plsc.subcore_barrier()
    pltpu.sync_copy(
        acc_sh.at[pl.ds(s * RPT, RPT)],
        out_hbm.at[c, pl.ds(s * RPT, RPT)],
    )


BR = 2048
NB = -(-N // BR)


def _scale_body(deg_ref, x_ref, w_ref, h0_ref, h1_ref, dinv_ref):
    deg = jnp.sum(deg_ref[...], axis=0) + 1.0
    dinv = lax.rsqrt(deg)[:, None]
    dinv_ref[...] = dinv
    h = jnp.dot(
        x_ref[...], w_ref[...],
        preferred_element_type=jnp.float32,
        precision=lax.Precision.HIGHEST,
    ) * dinv
    h0_ref[...] = h[:, :NH]
    h1_ref[...] = h[:, NH:]


_scale = pl.pallas_call(
    _scale_body,
    grid=(NB,),
    in_specs=[
        pl.BlockSpec((NC * NS, BR), lambda i: (0, i)),
        pl.BlockSpec((BR, D), lambda i: (i, 0)),
        pl.BlockSpec((D, D), lambda i: (0, 0)),
    ],
    out_specs=[
        pl.BlockSpec((BR, NH), lambda i: (i, 0)),
        pl.BlockSpec((BR, NH), lambda i: (i, 0)),
        pl.BlockSpec((BR, 1), lambda i: (i, 0)),
    ],
    out_shape=[
        jax.ShapeDtypeStruct((N, NH), jnp.float32),
        jax.ShapeDtypeStruct((N, NH), jnp.float32),
        jax.ShapeDtypeStruct((N, 1), jnp.float32),
    ],
)


def _mid_body(p_ref, h0_ref, h1_ref, dinv_ref, b1_ref, w2_ref,
              h20_ref, h21_ref):
    agg = jnp.concatenate(
        [p_ref[0] + h0_ref[...], p_ref[1] + h1_ref[...]],
        axis=1,
    )
    z = jnp.maximum(agg * dinv_ref[...] + b1_ref[...][None, :], 0.0)
    h2 = jnp.dot(
        z, w2_ref[...],
        preferred_element_type=jnp.float32,
        precision=lax.Precision.HIGHEST,
    ) * dinv_ref[...]
    h20_ref[...] = h2[:, :NH]
    h21_ref[...] = h2[:, NH:]


_mid = pl.pallas_call(
    _mid_body,
    grid=(NB,),
    in_specs=[
        pl.BlockSpec((NC, BR, NH), lambda i: (0, i, 0)),
        pl.BlockSpec((BR, NH), lambda i: (i, 0)),
        pl.BlockSpec((BR, NH), lambda i: (i, 0)),
        pl.BlockSpec((BR, 1), lambda i: (i, 0)),
        pl.BlockSpec((D,), lambda i: (0,)),
        pl.BlockSpec((D, D), lambda i: (0, 0)),
    ],
    out_specs=[
        pl.BlockSpec((BR, NH), lambda i: (i, 0)),
        pl.BlockSpec((BR, NH), lambda i: (i, 0)),
    ],
    out_shape=[
        jax.ShapeDtypeStruct((N, NH), jnp.float32),
        jax.ShapeDtypeStruct((N, NH), jnp.float32),
    ],
)


def _out_body(p_ref, h20_ref, h21_ref, dinv_ref, b2_ref, o_ref):
    agg = jnp.concatenate(
        [p_ref[0] + h20_ref[...], p_ref[1] + h21_ref[...]],
        axis=1,
    )
    o_ref[...] = agg * dinv_ref[...] + b2_ref[...][None, :]


_out = pl.pallas_call(
    _out_body,
    grid=(NB,),
    in_specs=[
        pl.BlockSpec((NC, BR, NH), lambda i: (0, i, 0)),
        pl.BlockSpec((BR, NH), lambda i: (i, 0)),
        pl.BlockSpec((BR, NH), lambda i: (i, 0)),
        pl.BlockSpec((BR, 1), lambda i: (i, 0)),
        pl.BlockSpec((D,), lambda i: (0,)),
    ],
    out_specs=pl.BlockSpec((BR, D), lambda i: (i, 0)),
    out_shape=jax.ShapeDtypeStruct((N, D), jnp.float32),
)


@jax.jit
def kernel(x, edge_index, W1, b1, W2, b2):
    src = edge_index[0].astype(jnp.int32)
    dst = edge_index[1].astype(jnp.int32)
    pad = EPAD - E
    srcp = jnp.concatenate([src, jnp.zeros((pad,), jnp.int32)])
    dstp = jnp.concatenate([dst, jnp.full((pad,), N, jnp.int32)])
    srcp16 = srcp.reshape(NS, CH, K)
    dstp16 = dstp.reshape(NS, CH, K)
    dstp32 = dstp.reshape(NC * NS, DCH, K)

    deg = _deg_kernel(dstp32)
    h0, h1, dinv = _scale(deg, x, W1)
    p1 = _agg_kernel(h0, h1, srcp16, dstp16)
    h20, h21 = _mid(p1, h0, h1, dinv, b1, W2)
    p2 = _agg_kernel(h20, h21, srcp16, dstp16)
    return _out(p2, h20, h21, dinv, b2)


# speedup vs baseline: 1.4791x; 1.2458x over previous
import dataclasses
import functools

import jax
import jax.numpy as jnp
from jax import lax
from jax.experimental import pallas as pl
from jax.experimental.pallas import tpu as pltpu
from jax.experimental.pallas import tpu_sc as plsc

N = 10000
E = 320000
D = 128
NH = 64
NC = 2
NS = 16
K = 128
FIRE = 2
IW = 20
CH = 160
EPAD = NS * CH * K
RPT = 640
RPAD = NS * RPT
DCH = 80
DEG_P = 10240
NWIN = CH // IW

_mesh = plsc.VectorSubcoreMesh(core_axis_name="c", subcore_axis_name="s")

_sc_params = pltpu.CompilerParams()
if "needs_layout_passes" in pltpu.CompilerParams.__dataclass_fields__:
    _sc_params = dataclasses.replace(_sc_params, needs_layout_passes=False)
if "use_tc_tiling_on_sc" in pltpu.CompilerParams.__dataclass_fields__:
    _sc_params = dataclasses.replace(_sc_params, use_tc_tiling_on_sc=False)


@functools.partial(
    pl.kernel,
    out_type=jax.ShapeDtypeStruct((NC * NS, DEG_P), jnp.float32),
    mesh=_mesh,
    compiler_params=_sc_params,
    scratch_types=[
        pltpu.VMEM((DCH, K), jnp.int32),
        pltpu.VMEM((DEG_P,), jnp.float32),
    ],
)
def _deg_kernel(dst_hbm, deg_hbm, dst_v, hist_v):
    w = lax.axis_index("c") * NS + lax.axis_index("s")
    pltpu.sync_copy(dst_hbm.at[w], dst_v)

    @pl.loop(0, DEG_P // 16)
    def _(i):
        hist_v[pl.ds(i * 16, 16)] = jnp.zeros((16,), jnp.float32)

    ones = jnp.ones((16,), jnp.float32)

    @pl.loop(0, DCH)
    def _(cc):
        for j in range(K // 16):
            idx = dst_v[cc, pl.ds(j * 16, 16)]
            plsc.addupdate_scatter(hist_v, [idx], ones)

    pltpu.sync_copy(hist_v, deg_hbm.at[w])


@functools.partial(
    pl.kernel,
    out_type=jax.ShapeDtypeStruct((NC, RPAD, NH), jnp.float32),
    mesh=_mesh,
    compiler_params=_sc_params,
    scratch_types=[
        pltpu.VMEM((3, IW, K), jnp.int32),
        pltpu.VMEM((3, IW, K), jnp.int32),
        pltpu.VMEM((2 * FIRE * K, NH), jnp.float32),
        pltpu.VMEM_SHARED((RPAD, NH), jnp.float32),
        pltpu.VMEM_SHARED((N, NH), jnp.float32),
        pltpu.SemaphoreType.DMA,
        pltpu.SemaphoreType.DMA,
        pltpu.SemaphoreType.DMA,
        pltpu.SemaphoreType.DMA,
    ],
)
def _agg_kernel(h0_hbm, h1_hbm, src_hbm, dst_hbm, out_hbm, src_v, dst_v,
                rows_v, acc_sh, h_sh, gsem, hsem, ssem0, ssem1):
    c = lax.axis_index("c")
    s = lax.axis_index("s")

    HRT = N // NS

    @pl.when(c == 0)
    def _():
        pltpu.sync_copy(h0_hbm.at[pl.ds(s * HRT, HRT)], h_sh.at[pl.ds(s * HRT, HRT)])

    @pl.when(c == 1)
    def _():
        pltpu.sync_copy(h1_hbm.at[pl.ds(s * HRT, HRT)], h_sh.at[pl.ds(s * HRT, HRT)])

    @pl.loop(0, 2 * FIRE * K)
    def _(r):
        for j in range(NH // 16):
            rows_v[r, pl.ds(j * 16, 16)] = jnp.zeros((16,), jnp.float32)

    pltpu.sync_copy(rows_v, acc_sh.at[pl.ds(s * RPT, 2 * FIRE * K)])
    pltpu.sync_copy(
        rows_v.at[pl.ds(0, RPT - 2 * FIRE * K)],
        acc_sh.at[pl.ds(s * RPT + 2 * FIRE * K, RPT - 2 * FIRE * K)],
    )
    plsc.subcore_barrier()

    pltpu.sync_copy(src_hbm.at[s, pl.ds(0, IW)], src_v.at[0])
    pltpu.sync_copy(dst_hbm.at[s, pl.ds(0, IW)], dst_v.at[0])
    for q in range(4):
        pltpu.async_copy(
            rows_v.at[pl.ds(q * K, K)],
            acc_sh.at[dst_v.at[0, q]],
            ssem0 if q < 2 else ssem1,
            add=True,
        )
    pending = None
    for t in range(NWIN):
        slot = t % 3
        if pending is not None:
            for cp in pending:
                cp.wait()
        if t + 1 < NWIN:
            nslot = (t + 1) % 3
            pending = [
                pltpu.async_copy(
                    src_hbm.at[s, pl.ds((t + 1) * IW, IW)], src_v.at[nslot],
                    hsem,
                ),
                pltpu.async_copy(
                    dst_hbm.at[s, pl.ds((t + 1) * IW, IW)], dst_v.at[nslot],
                    hsem,
                ),
            ]
        else:
            pending = None

        @pl.loop(0, IW, step=4)
        def _(cl):
            for sub in range(2):
                ssem = ssem0 if sub == 0 else ssem1
                pltpu.make_async_copy(
                    h0_hbm.at[pl.ds(0, 2 * K)],
                    rows_v.at[pl.ds(sub * 2 * K, 2 * K)],
                    ssem,
                ).wait()
                g = []
                for i in range(2):
                    q = sub * 2 + i
                    g.append(
                        pltpu.async_copy(
                            h_sh.at[src_v.at[slot, cl + q]],
                            rows_v.at[pl.ds(q * K, K)],
                            gsem,
                        )
                    )
                for cp in g:
                    cp.wait()
                for i in range(2):
                    q = sub * 2 + i
                    pltpu.async_copy(
                        rows_v.at[pl.ds(q * K, K)],
                        acc_sh.at[dst_v.at[slot, cl + q]],
                        ssem,
                        add=True,
                    )

    pltpu.make_async_copy(
        h0_hbm.at[pl.ds(0, 2 * K)],
        rows_v.at[pl.ds(0, 2 * K)],
        ssem0,
    ).wait()
    pltpu.make_async_copy(
        h0_hbm.at[pl.ds(0, 2 * K)],
        rows_v.at[pl.ds(2 * K, 2 * K)],
        ssem1,
    ).wait()
    plsc.subcore_barrier()
    pltpu.sync_copy(
        acc_sh.at[pl.ds(s * RPT, RPT)],
        out_hbm.at[c, pl.ds(s * RPT, RPT)],
    )


BR = 2048
NB = -(-N // BR)


def _scale_body(deg_ref, x_ref, w_ref, h0_ref, h1_ref, dinv_ref):
    deg = jnp.sum(deg_ref[...], axis=0) + 1.0
    dinv = lax.rsqrt(deg)[:, None]
    dinv_ref[...] = dinv
    h = jnp.dot(
        x_ref[...], w_ref[...],
        preferred_element_type=jnp.float32,
        precision=lax.Precision.HIGHEST,
    ) * dinv
    h0_ref[...] = h[:, :NH]
    h1_ref[...] = h[:, NH:]


_scale = pl.pallas_call(
    _scale_body,
    grid=(NB,),
    in_specs=[
        pl.BlockSpec((NC * NS, BR), lambda i: (0, i)),
        pl.BlockSpec((BR, D), lambda i: (i, 0)),
        pl.BlockSpec((D, D), lambda i: (0, 0)),
    ],
    out_specs=[
        pl.BlockSpec((BR, NH), lambda i: (i, 0)),
        pl.BlockSpec((BR, NH), lambda i: (i, 0)),
        pl.BlockSpec((BR, 1), lambda i: (i, 0)),
    ],
    out_shape=[
        jax.ShapeDtypeStruct((N, NH), jnp.float32),
        jax.ShapeDtypeStruct((N, NH), jnp.float32),
        jax.ShapeDtypeStruct((N, 1), jnp.float32),
    ],
)


def _mid_body(p_ref, h0_ref, h1_ref, dinv_ref, b1_ref, w2_ref,
              h20_ref, h21_ref):
    agg = jnp.concatenate(
        [p_ref[0] + h0_ref[...], p_ref[1] + h1_ref[...]],
        axis=1,
    )
    z = jnp.maximum(agg * dinv_ref[...] + b1_ref[...][None, :], 0.0)
    h2 = jnp.dot(
        z, w2_ref[...],
        preferred_element_type=jnp.float32,
        precision=lax.Precision.HIGHEST,
    ) * dinv_ref[...]
    h20_ref[...] = h2[:, :NH]
    h21_ref[...] = h2[:, NH:]


_mid = pl.pallas_call(
    _mid_body,
    grid=(NB,),
    in_specs=[
        pl.BlockSpec((NC, BR, NH), lambda i: (0, i, 0)),
        pl.BlockSpec((BR, NH), lambda i: (i, 0)),
        pl.BlockSpec((BR, NH), lambda i: (i, 0)),
        pl.BlockSpec((BR, 1), lambda i: (i, 0)),
        pl.BlockSpec((D,), lambda i: (0,)),
        pl.BlockSpec((D, D), lambda i: (0, 0)),
    ],
    out_specs=[
        pl.BlockSpec((BR, NH), lambda i: (i, 0)),
        pl.BlockSpec((BR, NH), lambda i: (i, 0)),
    ],
    out_shape=[
        jax.ShapeDtypeStruct((N, NH), jnp.float32),
        jax.ShapeDtypeStruct((N, NH), jnp.float32),
    ],
)


def _out_body(p_ref, h20_ref, h21_ref, dinv_ref, b2_ref, o_ref):
    agg = jnp.concatenate(
        [p_ref[0] + h20_ref[...], p_ref[1] + h21_ref[...]],
        axis=1,
    )
    o_ref[...] = agg * dinv_ref[...] + b2_ref[...][None, :]


_out = pl.pallas_call(
    _out_body,
    grid=(NB,),
    in_specs=[
        pl.BlockSpec((NC, BR, NH), lambda i: (0, i, 0)),
        pl.BlockSpec((BR, NH), lambda i: (i, 0)),
        pl.BlockSpec((BR, NH), lambda i: (i, 0)),
        pl.BlockSpec((BR, 1), lambda i: (i, 0)),
        pl.BlockSpec((D,), lambda i: (0,)),
    ],
    out_specs=pl.BlockSpec((BR, D), lambda i: (i, 0)),
    out_shape=jax.ShapeDtypeStruct((N, D), jnp.float32),
)


@jax.jit
def kernel(x, edge_index, W1, b1, W2, b2):
    src = edge_index[0].astype(jnp.int32)
    dst = edge_index[1].astype(jnp.int32)
    pad = EPAD - E
    srcp = jnp.concatenate([src, jnp.zeros((pad,), jnp.int32)])
    dstp = jnp.concatenate([dst, jnp.full((pad,), N, jnp.int32)])
    srcp16 = srcp.reshape(NS, CH, K)
    dstp16 = dstp.reshape(NS, CH, K)
    dstp32 = dstp.reshape(NC * NS, DCH, K)

    deg = _deg_kernel(dstp32)
    h0, h1, dinv = _scale(deg, x, W1)
    p1 = _agg_kernel(h0, h1, srcp16, dstp16)
    h20, h21 = _mid(p1, h0, h1, dinv, b1, W2)
    p2 = _agg_kernel(h20, h21, srcp16, dstp16)
    return _out(p2, h20, h21, dinv, b2)


# speedup vs baseline: 1.4824x; 1.0023x over previous
import dataclasses
import functools

import jax
import jax.numpy as jnp
from jax import lax
from jax.experimental import pallas as pl
from jax.experimental.pallas import tpu as pltpu
from jax.experimental.pallas import tpu_sc as plsc

N = 10000
E = 320000
D = 128
NH = 64
NC = 2
NS = 16
K = 128
FIRE = 2
IW = 20
CH = 160
EPAD = NS * CH * K
RPT = 640
RPAD = NS * RPT
DCH = 80
DEG_P = 10240
NWIN = CH // IW

_mesh = plsc.VectorSubcoreMesh(core_axis_name="c", subcore_axis_name="s")

_sc_params = pltpu.CompilerParams()
if "needs_layout_passes" in pltpu.CompilerParams.__dataclass_fields__:
    _sc_params = dataclasses.replace(_sc_params, needs_layout_passes=False)
if "use_tc_tiling_on_sc" in pltpu.CompilerParams.__dataclass_fields__:
    _sc_params = dataclasses.replace(_sc_params, use_tc_tiling_on_sc=False)


@functools.partial(
    pl.kernel,
    out_type=jax.ShapeDtypeStruct((NC * NS, DEG_P), jnp.float32),
    mesh=_mesh,
    compiler_params=_sc_params,
    scratch_types=[
        pltpu.VMEM((DCH, K), jnp.int32),
        pltpu.VMEM((DEG_P,), jnp.float32),
    ],
)
def _deg_kernel(dst_hbm, deg_hbm, dst_v, hist_v):
    w = lax.axis_index("c") * NS + lax.axis_index("s")
    pltpu.sync_copy(dst_hbm.at[w], dst_v)

    @pl.loop(0, DEG_P // 16)
    def _(i):
        hist_v[pl.ds(i * 16, 16)] = jnp.zeros((16,), jnp.float32)

    ones = jnp.ones((16,), jnp.float32)

    @pl.loop(0, DCH)
    def _(cc):
        for j in range(K // 16):
            idx = dst_v[cc, pl.ds(j * 16, 16)]
            plsc.addupdate_scatter(hist_v, [idx], ones)

    pltpu.sync_copy(hist_v, deg_hbm.at[w])


@functools.partial(
    pl.kernel,
    out_type=jax.ShapeDtypeStruct((NC, RPAD, NH), jnp.float32),
    mesh=_mesh,
    compiler_params=_sc_params,
    scratch_types=[
        pltpu.VMEM((3, IW, K), jnp.int32),
        pltpu.VMEM((3, IW, K), jnp.int32),
        pltpu.VMEM((2 * FIRE * K, NH), jnp.float32),
        pltpu.VMEM_SHARED((RPAD, NH), jnp.float32),
        pltpu.VMEM_SHARED((N, NH), jnp.float32),
        pltpu.SemaphoreType.DMA,
        pltpu.SemaphoreType.DMA,
        pltpu.SemaphoreType.DMA,
        pltpu.SemaphoreType.DMA,
    ],
)
def _agg_kernel(h0_hbm, h1_hbm, src_hbm, dst_hbm, out_hbm, src_v, dst_v,
                rows_v, acc_sh, h_sh, gsem, hsem, ssem0, ssem1):
    c = lax.axis_index("c")
    s = lax.axis_index("s")

    HRT = N // NS

    @pl.when(c == 0)
    def _():
        pltpu.sync_copy(h0_hbm.at[pl.ds(s * HRT, HRT)], h_sh.at[pl.ds(s * HRT, HRT)])

    @pl.when(c == 1)
    def _():
        pltpu.sync_copy(h1_hbm.at[pl.ds(s * HRT, HRT)], h_sh.at[pl.ds(s * HRT, HRT)])

    @pl.loop(0, 2 * FIRE * K)
    def _(r):
        for j in range(NH // 16):
            rows_v[r, pl.ds(j * 16, 16)] = jnp.zeros((16,), jnp.float32)

    pltpu.sync_copy(rows_v, acc_sh.at[pl.ds(s * RPT, 2 * FIRE * K)])
    pltpu.sync_copy(
        rows_v.at[pl.ds(0, RPT - 2 * FIRE * K)],
        acc_sh.at[pl.ds(s * RPT + 2 * FIRE * K, RPT - 2 * FIRE * K)],
    )
    plsc.subcore_barrier()

    pltpu.sync_copy(src_hbm.at[s, pl.ds(0, IW)], src_v.at[0])
    pltpu.sync_copy(dst_hbm.at[s, pl.ds(0, IW)], dst_v.at[0])
    for q in range(4):
        pltpu.async_copy(
            rows_v.at[pl.ds(q * K, K)],
            acc_sh.at[dst_v.at[0, q]],
            ssem0 if q < 2 else ssem1,
            add=True,
        )
    pending = None
    for t in range(NWIN):
        slot = t % 3
        if pending is not None:
            for cp in pending:
                cp.wait()
        if t + 1 < NWIN:
            nslot = (t + 1) % 3
            pending = [
                pltpu.async_copy(
                    src_hbm.at[s, pl.ds((t + 1) * IW, IW)], src_v.at[nslot],
                    hsem,
                ),
                pltpu.async_copy(
                    dst_hbm.at[s, pl.ds((t + 1) * IW, IW)], dst_v.at[nslot],
                    hsem,
                ),
            ]
        else:
            pending = None

        @pl.loop(0, IW, step=4)
        def _(cl):
            for sub in range(2):
                ssem = ssem0 if sub == 0 else ssem1
                pltpu.make_async_copy(
                    h0_hbm.at[pl.ds(0, 2 * K)],
                    rows_v.at[pl.ds(sub * 2 * K, 2 * K)],
                    ssem,
                ).wait()
                g = []
                for i in range(2):
                    q = sub * 2 + i
                    g.append(
                        pltpu.async_copy(
                            h_sh.at[src_v.at[slot, cl + q]],
                            rows_v.at[pl.ds(q * K, K)],
                            gsem,
                        )
                    )
                for cp in g:
                    cp.wait()
                for i in range(2):
                    q = sub * 2 + i
                    pltpu.async_copy(
                        rows_v.at[pl.ds(q * K, K)],
                        acc_sh.at[dst_v.at[slot, cl + q]],
                        ssem,
                        add=True,
                    )

    pltpu.make_async_copy(
        h0_hbm.at[pl.ds(0, 2 * K)],
        rows_v.at[pl.ds(0, 2 * K)],
        ssem0,
    ).wait()
    pltpu.make_async_copy(
        h0_hbm.at[pl.ds(0, 2 * K)],
        rows_v.at[pl.ds(2 * K, 2 * K)],
        ssem1,
    ).wait()
    plsc.subcore_barrier()
    pltpu.sync_copy(
        acc_sh.at[pl.ds(s * RPT, RPT)],
        out_hbm.at[c, pl.ds(s * RPT, RPT)],
    )


BR = 2048
NB = -(-N // BR)


def _mm1_body(x_ref, w_ref, h_ref):
    h_ref[...] = jnp.dot(
        x_ref[...], w_ref[...],
        preferred_element_type=jnp.float32,
        precision=lax.Precision.HIGHEST,
    )


_mm1 = pl.pallas_call(
    _mm1_body,
    grid=(NB,),
    in_specs=[
        pl.BlockSpec((BR, D), lambda i: (i, 0)),
        pl.BlockSpec((D, D), lambda i: (0, 0)),
    ],
    out_specs=pl.BlockSpec((BR, D), lambda i: (i, 0)),
    out_shape=jax.ShapeDtypeStruct((N, D), jnp.float32),
)


def _scale_body(deg_ref, h_ref, h0_ref, h1_ref, dinv_ref):
    deg = jnp.sum(deg_ref[...], axis=0) + 1.0
    dinv = lax.rsqrt(deg)[:, None]
    dinv_ref[...] = dinv
    h = h_ref[...] * dinv
    h0_ref[...] = h[:, :NH]
    h1_ref[...] = h[:, NH:]


_scale = pl.pallas_call(
    _scale_body,
    grid=(NB,),
    in_specs=[
        pl.BlockSpec((NC * NS, BR), lambda i: (0, i)),
        pl.BlockSpec((BR, D), lambda i: (i, 0)),
    ],
    out_specs=[
        pl.BlockSpec((BR, NH), lambda i: (i, 0)),
        pl.BlockSpec((BR, NH), lambda i: (i, 0)),
        pl.BlockSpec((BR, 1), lambda i: (i, 0)),
    ],
    out_shape=[
        jax.ShapeDtypeStruct((N, NH), jnp.float32),
        jax.ShapeDtypeStruct((N, NH), jnp.float32),
        jax.ShapeDtypeStruct((N, 1), jnp.float32),
    ],
)


def _mid_body(p_ref, h0_ref, h1_ref, dinv_ref, b1_ref, w2_ref,
              h20_ref, h21_ref):
    agg = jnp.concatenate(
        [p_ref[0] + h0_ref[...], p_ref[1] + h1_ref[...]],
        axis=1,
    )
    z = jnp.maximum(agg * dinv_ref[...] + b1_ref[...][None, :], 0.0)
    h2 = jnp.dot(
        z, w2_ref[...],
        preferred_element_type=jnp.float32,
        precision=lax.Precision.HIGHEST,
    ) * dinv_ref[...]
    h20_ref[...] = h2[:, :NH]
    h21_ref[...] = h2[:, NH:]


_mid = pl.pallas_call(
    _mid_body,
    grid=(NB,),
    in_specs=[
        pl.BlockSpec((NC, BR, NH), lambda i: (0, i, 0)),
        pl.BlockSpec((BR, NH), lambda i: (i, 0)),
        pl.BlockSpec((BR, NH), lambda i: (i, 0)),
        pl.BlockSpec((BR, 1), lambda i: (i, 0)),
        pl.BlockSpec((D,), lambda i: (0,)),
        pl.BlockSpec((D, D), lambda i: (0, 0)),
    ],
    out_specs=[
        pl.BlockSpec((BR, NH), lambda i: (i, 0)),
        pl.BlockSpec((BR, NH), lambda i: (i, 0)),
    ],
    out_shape=[
        jax.ShapeDtypeStruct((N, NH), jnp.float32),
        jax.ShapeDtypeStruct((N, NH), jnp.float32),
    ],
)


def _out_body(p_ref, h20_ref, h21_ref, dinv_ref, b2_ref, o_ref):
    agg = jnp.concatenate(
        [p_ref[0] + h20_ref[...], p_ref[1] + h21_ref[...]],
        axis=1,
    )
    o_ref[...] = agg * dinv_ref[...] + b2_ref[...][None, :]


_out = pl.pallas_call(
    _out_body,
    grid=(NB,),
    in_specs=[
        pl.BlockSpec((NC, BR, NH), lambda i: (0, i, 0)),
        pl.BlockSpec((BR, NH), lambda i: (i, 0)),
        pl.BlockSpec((BR, NH), lambda i: (i, 0)),
        pl.BlockSpec((BR, 1), lambda i: (i, 0)),
        pl.BlockSpec((D,), lambda i: (0,)),
    ],
    out_specs=pl.BlockSpec((BR, D), lambda i: (i, 0)),
    out_shape=jax.ShapeDtypeStruct((N, D), jnp.float32),
)


@jax.jit
def kernel(x, edge_index, W1, b1, W2, b2):
    src = edge_index[0].astype(jnp.int32)
    dst = edge_index[1].astype(jnp.int32)
    pad = EPAD - E
    srcp = jnp.concatenate([src, jnp.zeros((pad,), jnp.int32)])
    dstp = jnp.concatenate([dst, jnp.full((pad,), N, jnp.int32)])
    srcp16 = srcp.reshape(NS, CH, K)
    dstp16 = dstp.reshape(NS, CH, K)
    dstp32 = dstp.reshape(NC * NS, DCH, K)

    deg = _deg_kernel(dstp32)
    h = _mm1(x, W1)
    h0, h1, dinv = _scale(deg, h)
    p1 = _agg_kernel(h0, h1, srcp16, dstp16)
    h20, h21 = _mid(p1, h0, h1, dinv, b1, W2)
    p2 = _agg_kernel(h20, h21, srcp16, dstp16)
    return _out(p2, h20, h21, dinv, b2)
